# Initial kernel scaffold; baseline (speedup 1.0000x reference)
import jax
import jax.numpy as jnp
from jax.experimental import pallas as pl


def kernel(x, edge_index, edge_attr, edge_type, bias_matrix, We, be, W1, b1, W2, b2, etw, Wq, bq, Wk, bk, Wv, bv, Wo, bo, F1, f1b, F2, f2b, gamma, beta):
    raise NotImplementedError("write your pallas kernel here")


# speedup vs baseline: 4.8267x; 4.8267x over previous
import functools

import jax
import jax.numpy as jnp
from jax import lax
from jax.experimental import pallas as pl
from jax.experimental.pallas import tpu as pltpu
from jax.experimental.pallas import tpu_sc as plsc

N = 2048
E = 262144
H = 128
NH = 8
DH = H // NH
ET = 4
ED = 6

F32 = jnp.float32

NC = 2
NS = 16
NW = NC * NS
EDGES_PER_W = E // NW
CHUNK = 128
NCHUNK = EDGES_PER_W // CHUNK
ROWS = ET * N
ROWS_PER_TILE = ROWS // NS

_DN = (((1,), (1,)), ((), ()))
_DN_STD = (((1,), (0,)), ((), ()))


def _dot_t(a, b):
    return lax.dot_general(a, b, _DN, preferred_element_type=F32)


def _dot(a, b):
    return lax.dot_general(a, b, _DN_STD, preferred_element_type=F32)


def _emb_body(attr_ref, oh_ref, we_ref, be_ref, out_ref):
    attr = attr_ref[...].astype(jnp.bfloat16)
    oh = oh_ref[...]
    acc = lax.dot_general(oh, be_ref[...], _DN_STD, preferred_element_type=F32,
                          precision=lax.Precision.HIGHEST)
    for t in range(ET):
        et = lax.dot_general(attr, we_ref[t].astype(jnp.bfloat16), _DN,
                             preferred_element_type=F32)
        acc = acc + oh[:, t:t + 1] * et
    out_ref[...] = acc


def _run_emb(edge_attr, oh, we, be, *, interpret=False):
    BE = 2048
    grid = (E // BE,)
    return pl.pallas_call(
        _emb_body,
        grid=grid,
        in_specs=[
            pl.BlockSpec((BE, ED), lambda i: (i, 0)),
            pl.BlockSpec((BE, ET), lambda i: (i, 0)),
            pl.BlockSpec((ET, H, ED), lambda i: (0, 0, 0)),
            pl.BlockSpec((ET, H), lambda i: (0, 0)),
        ],
        out_specs=pl.BlockSpec((BE, H), lambda i: (i, 0)),
        out_shape=jax.ShapeDtypeStruct((E, H), F32),
        interpret=interpret,
    )(edge_attr, oh, we, be)


def _edge_sc_body(x_hbm, src_hbm, tidx_hbm, emb_hbm, out_hbm,
                  src_v, tidx_v, emb_v, xrows_v, aggr_s, sem):
    cid = lax.axis_index("c")
    sid = lax.axis_index("s")
    wid = sid * NC + cid
    base = wid * EDGES_PER_W

    def _z(j, _):
        for k in range(H // 16):
            xrows_v[j, pl.ds(k * 16, 16)] = jnp.zeros((16,), F32)
        return 0
    lax.fori_loop(0, CHUNK, _z, 0)
    for r in range(ROWS_PER_TILE // CHUNK):
        pltpu.sync_copy(xrows_v, aggr_s.at[pl.ds(sid * ROWS_PER_TILE + r * CHUNK, CHUNK)])
    plsc.subcore_barrier()

    def _chunk(g, _):
        off = pl.multiple_of(base + g * CHUNK, CHUNK)
        pltpu.sync_copy(src_hbm.at[pl.ds(off, CHUNK)], src_v)
        pltpu.sync_copy(tidx_hbm.at[pl.ds(off, CHUNK)], tidx_v)
        pltpu.sync_copy(emb_hbm.at[pl.ds(off, CHUNK)], emb_v)
        pltpu.async_copy(x_hbm.at[src_v], xrows_v, sem).wait()

        def _edge(j, _):
            for k in range(H // 16):
                s = pl.ds(k * 16, 16)
                emb_v[j, s] = jnp.maximum(emb_v[j, s] + xrows_v[j, s], 0.0)
            return 0
        lax.fori_loop(0, CHUNK, _edge, 0)

        pltpu.sync_copy(emb_v, aggr_s.at[tidx_v], add=True)
        return 0

    lax.fori_loop(0, NCHUNK, _chunk, 0)
    plsc.subcore_barrier()

    row0 = cid * ROWS + sid * ROWS_PER_TILE
    pltpu.sync_copy(aggr_s.at[pl.ds(sid * ROWS_PER_TILE, ROWS_PER_TILE)],
                    out_hbm.at[pl.ds(row0, ROWS_PER_TILE)])


def _run_edge_sc(x, src, tidx, emb):
    mesh = plsc.VectorSubcoreMesh(core_axis_name="c", subcore_axis_name="s")
    fn = pl.kernel(
        _edge_sc_body,
        mesh=mesh,
        out_type=jax.ShapeDtypeStruct((NC * ROWS, H), F32),
        scratch_types=[
            pltpu.VMEM((CHUNK,), jnp.int32),
            pltpu.VMEM((CHUNK,), jnp.int32),
            pltpu.VMEM((CHUNK, H), F32),
            pltpu.VMEM((CHUNK, H), F32),
            pltpu.VMEM_SHARED((ROWS, H), F32),
            pltpu.SemaphoreType.DMA,
        ],
    )
    return fn(x, src, tidx, emb)


def _qkv_body(x_ref, w_ref, b_ref, out_ref):
    out_ref[...] = _dot_t(x_ref[...], w_ref[...]) + b_ref[...]


def _run_qkv(x, wqkv, bqkv, *, interpret=False):
    return pl.pallas_call(
        _qkv_body,
        out_shape=jax.ShapeDtypeStruct((N, 3 * H), F32),
        interpret=interpret,
    )(x, wqkv, bqkv)


def _attn_body(q_ref, k_ref, v_ref, bias_ref, out_ref):
    bias = bias_ref[...]
    cols = []
    for h in range(NH):
        s = slice(h * DH, (h + 1) * DH)
        qh = q_ref[:, s] * (1.0 / (DH ** 0.5))
        sc = _dot_t(qh, k_ref[:, s]) + bias
        m = jnp.max(sc, axis=1, keepdims=True)
        e = jnp.exp(sc - m)
        den = jnp.sum(e, axis=1, keepdims=True)
        p = e / den
        cols.append(_dot(p, v_ref[:, s]))
    out_ref[...] = jnp.concatenate(cols, axis=1)


def _run_attn(q, k, v, bias, *, interpret=False):
    BQ = 256
    grid = (N // BQ,)
    return pl.pallas_call(
        _attn_body,
        grid=grid,
        in_specs=[
            pl.BlockSpec((BQ, H), lambda i: (i, 0)),
            pl.BlockSpec((N, H), lambda i: (0, 0)),
            pl.BlockSpec((N, H), lambda i: (0, 0)),
            pl.BlockSpec((BQ, N), lambda i: (i, 0)),
        ],
        out_specs=pl.BlockSpec((BQ, H), lambda i: (i, 0)),
        out_shape=jax.ShapeDtypeStruct((N, H), F32),
        interpret=interpret,
    )(q, k, v, bias)


def _combine_body(x_ref, sc_ref, o_ref, w1_ref, b1_ref, w2_ref, b2_ref,
                  etw_ref, wo_ref, bo_ref, f1_ref, f1b_ref, f2_ref, f2b_ref,
                  gamma_ref, beta_ref, out_ref):
    x = x_ref[...]
    aggr = sc_ref[0:ROWS, :] + sc_ref[ROWS:2 * ROWS, :]

    etw = etw_ref[...]
    ew = jnp.exp(etw - jnp.max(etw))
    w = ew / jnp.sum(ew)

    xm = jnp.zeros((N, H), F32)
    for t in range(ET):
        xa = x + aggr[t * N:(t + 1) * N, :]
        h1 = jnp.maximum(_dot_t(xa, w1_ref[t]) + b1_ref[t:t + 1, :], 0.0)
        ht = _dot_t(h1, w2_ref[t]) + b2_ref[t:t + 1, :]
        xm = xm + w[0:1, t:t + 1] * ht

    xat = _dot_t(o_ref[...], wo_ref[...]) + bo_ref[...]
    cat = jnp.concatenate([xm, xat], axis=1)
    h = jnp.maximum(_dot_t(cat, f1_ref[...]) + f1b_ref[...], 0.0)
    f = _dot_t(h, f2_ref[...]) + f2b_ref[...]
    mu = jnp.mean(f, axis=0, keepdims=True)
    d = f - mu
    var = jnp.mean(d * d, axis=0, keepdims=True)
    fn = d * lax.rsqrt(var + 1e-5) * gamma_ref[...] + beta_ref[...]
    out_ref[...] = x + fn


def _run_combine(x, scout, o, w1, b1, w2, b2, etw2, wo, bo2,
                 f1, f1b2, f2, f2b2, gamma2, beta2, *, interpret=False):
    return pl.pallas_call(
        _combine_body,
        out_shape=jax.ShapeDtypeStruct((N, H), F32),
        interpret=interpret,
    )(x, scout, o, w1, b1, w2, b2, etw2, wo, bo2, f1, f1b2, f2, f2b2,
      gamma2, beta2)


def kernel(x, edge_index, edge_attr, edge_type, bias_matrix,
           We, be, W1, b1, W2, b2, etw,
           Wq, bq, Wk, bk, Wv, bv, Wo, bo,
           F1, f1b, F2, f2b, gamma, beta):
    src = edge_index[0]
    dst = edge_index[1]
    tidx = edge_type * N + dst
    oh = (edge_type[:, None] == jnp.arange(ET, dtype=jnp.int32)[None, :]).astype(F32)

    emb = _run_emb(edge_attr, oh, We, be)
    scout = _run_edge_sc(x, src, tidx, emb)

    wqkv = jnp.concatenate([Wq, Wk, Wv], axis=0)
    bqkv = jnp.concatenate([bq, bk, bv])[None, :]
    qkv = _run_qkv(x, wqkv, bqkv)
    q = qkv[:, 0:H]
    k = qkv[:, H:2 * H]
    v = qkv[:, 2 * H:3 * H]
    o = _run_attn(q, k, v, bias_matrix)

    out = _run_combine(
        x, scout, o, W1, b1, W2, b2, etw[None, :], Wo, bo[None, :],
        F1, f1b[None, :], F2, f2b[None, :], gamma[None, :], beta[None, :])
    return out


# speedup vs baseline: 5.9573x; 1.2342x over previous
import jax
import jax.numpy as jnp
from jax import lax
from jax.experimental import pallas as pl
from jax.experimental.pallas import tpu as pltpu
from jax.experimental.pallas import tpu_sc as plsc

N = 2048
E = 262144
H = 128
NH = 8
DH = H // NH
ET = 4
ED = 6

F32 = jnp.float32
BF16 = jnp.bfloat16

NC = 2
NS = 16
NW = NC * NS
EDGES_PER_W = E // NW
CHUNK = 64
NCHUNK = EDGES_PER_W // CHUNK
NPAIR = NCHUNK // 2
ROWS = ET * N
ROWS_PER_TILE = ROWS // NS

_DN = (((1,), (1,)), ((), ()))
_DN_STD = (((1,), (0,)), ((), ()))


def _dot_t(a, b):
    return lax.dot_general(a, b, _DN, preferred_element_type=F32)


def _dot(a, b):
    return lax.dot_general(a, b, _DN_STD, preferred_element_type=F32)


def _emb_body(attr_ref, et_ref, t24_ref, s24_ref, w24_ref, be_ref, out_ref):
    attr = attr_ref[...].astype(BF16)
    iot = lax.broadcasted_iota(jnp.int32, (1, ET), 1)
    oh = (et_ref[...] == iot).astype(F32)
    atiled = _dot(attr, t24_ref[...].astype(BF16))
    ohs = _dot(oh, s24_ref[...])
    feat = (atiled * ohs).astype(BF16)
    out_ref[...] = (_dot(feat, w24_ref[...].astype(BF16))
                    + lax.dot_general(oh, be_ref[...], _DN_STD,
                                      preferred_element_type=F32,
                                      precision=lax.Precision.HIGHEST))


def _run_emb(edge_attr, etcol, t24, s24, w24, be, *, interpret=False):
    BE = 2048
    grid = (E // BE,)
    return pl.pallas_call(
        _emb_body,
        grid=grid,
        in_specs=[
            pl.BlockSpec((BE, ED), lambda i: (i, 0)),
            pl.BlockSpec((BE, 1), lambda i: (i, 0)),
            pl.BlockSpec((ED, ET * ED), lambda i: (0, 0)),
            pl.BlockSpec((ET, ET * ED), lambda i: (0, 0)),
            pl.BlockSpec((ET * ED, H), lambda i: (0, 0)),
            pl.BlockSpec((ET, H), lambda i: (0, 0)),
        ],
        out_specs=pl.BlockSpec((BE, H), lambda i: (i, 0)),
        out_shape=jax.ShapeDtypeStruct((E, H), F32),
        interpret=interpret,
    )(edge_attr, etcol, t24, s24, w24, be)


def _relu_add_chunk(emb_v, xr_v):
    UN = 4

    def _grp(i, _):
        for u in range(UN):
            j = i * UN + u
            for k in range(H // 16):
                s = pl.ds(k * 16, 16)
                emb_v[j, s] = jnp.maximum(emb_v[j, s] + xr_v[j, s], 0.0)
        return 0
    lax.fori_loop(0, CHUNK // UN, _grp, 0)


def _edge_sc_body(x_hbm, srcr_hbm, tidxr_hbm, emb_hbm, out_hbm,
                  src_a, tidx_a, emb0, emb1, xr0, xr1, aggr_s,
                  se0, se1, sg0, sg1, ss0, ss1):
    cid = lax.axis_index("c")
    sid = lax.axis_index("s")
    wid = sid * NC + cid
    base = wid * EDGES_PER_W

    pltpu.sync_copy(srcr_hbm.at[wid], src_a)
    pltpu.sync_copy(tidxr_hbm.at[wid], tidx_a)

    def _z(j, _):
        for k in range(H // 16):
            emb0[j, pl.ds(k * 16, 16)] = jnp.zeros((16,), F32)
        return 0
    lax.fori_loop(0, CHUNK, _z, 0)
    for r in range(ROWS_PER_TILE // CHUNK):
        pltpu.sync_copy(emb0, aggr_s.at[pl.ds(sid * ROWS_PER_TILE + r * CHUNK, CHUNK)])
    plsc.subcore_barrier()

    def _start(g, ebuf, xbuf, esem, gsem):
        off = pl.multiple_of(base + g * CHUNK, CHUNK)
        ce = pltpu.async_copy(emb_hbm.at[pl.ds(off, CHUNK)], ebuf, esem)
        cg = pltpu.async_copy(x_hbm.at[src_a.at[g]], xbuf, gsem)
        del ce, cg

    _start(0, emb0, xr0, se0, sg0)
    _start(1, emb1, xr1, se1, sg1)

    def _pair(p, _):
        g0 = p * 2

        pltpu.make_async_copy(emb_hbm.at[pl.ds(0, CHUNK)], emb0, se0).wait()
        pltpu.make_async_copy(x_hbm.at[src_a.at[0]], xr0, sg0).wait()
        _relu_add_chunk(emb0, xr0)
        c0 = pltpu.async_copy(emb0, aggr_s.at[tidx_a.at[g0]], ss0, add=True)
        del c0

        pltpu.make_async_copy(emb_hbm.at[pl.ds(0, CHUNK)], emb1, se1).wait()
        pltpu.make_async_copy(x_hbm.at[src_a.at[0]], xr1, sg1).wait()
        _relu_add_chunk(emb1, xr1)
        c1 = pltpu.async_copy(emb1, aggr_s.at[tidx_a.at[g0 + 1]], ss1, add=True)
        del c1

        @pl.when(p < NPAIR - 1)
        def _():
            pltpu.make_async_copy(emb0, aggr_s.at[tidx_a.at[0]], ss0).wait()
            _start(g0 + 2, emb0, xr0, se0, sg0)
            pltpu.make_async_copy(emb1, aggr_s.at[tidx_a.at[0]], ss1).wait()
            _start(g0 + 3, emb1, xr1, se1, sg1)
        return 0

    lax.fori_loop(0, NPAIR, _pair, 0)
    pltpu.make_async_copy(emb0, aggr_s.at[tidx_a.at[0]], ss0).wait()
    pltpu.make_async_copy(emb1, aggr_s.at[tidx_a.at[0]], ss1).wait()
    plsc.subcore_barrier()

    row0 = cid * ROWS + sid * ROWS_PER_TILE
    pltpu.sync_copy(aggr_s.at[pl.ds(sid * ROWS_PER_TILE, ROWS_PER_TILE)],
                    out_hbm.at[pl.ds(row0, ROWS_PER_TILE)])


def _run_edge_sc(x, srcr, tidxr, emb):
    mesh = plsc.VectorSubcoreMesh(core_axis_name="c", subcore_axis_name="s")
    fn = pl.kernel(
        _edge_sc_body,
        mesh=mesh,
        out_type=jax.ShapeDtypeStruct((NC * ROWS, H), F32),
        scratch_types=[
            pltpu.VMEM((NCHUNK, CHUNK), jnp.int32),
            pltpu.VMEM((NCHUNK, CHUNK), jnp.int32),
            pltpu.VMEM((CHUNK, H), F32),
            pltpu.VMEM((CHUNK, H), F32),
            pltpu.VMEM((CHUNK, H), F32),
            pltpu.VMEM((CHUNK, H), F32),
            pltpu.VMEM_SHARED((ROWS, H), F32),
            pltpu.SemaphoreType.DMA,
            pltpu.SemaphoreType.DMA,
            pltpu.SemaphoreType.DMA,
            pltpu.SemaphoreType.DMA,
            pltpu.SemaphoreType.DMA,
            pltpu.SemaphoreType.DMA,
        ],
    )
    return fn(x, srcr, tidxr, emb)


def _qkv_body(x_ref, w_ref, b_ref, out_ref):
    out_ref[...] = _dot_t(x_ref[...], w_ref[...]) + b_ref[...]


def _run_qkv(x, wqkv, bqkv, *, interpret=False):
    return pl.pallas_call(
        _qkv_body,
        out_shape=jax.ShapeDtypeStruct((N, 3 * H), F32),
        interpret=interpret,
    )(x, wqkv, bqkv)


def _attn_body(q_ref, k_ref, v_ref, bias_ref, out_ref):
    bias = bias_ref[...]
    cols = []
    for h in range(NH):
        s = slice(h * DH, (h + 1) * DH)
        qh = q_ref[:, s] * (1.0 / (DH ** 0.5))
        sc = _dot_t(qh, k_ref[:, s]) + bias
        m = jnp.max(sc, axis=1, keepdims=True)
        e = jnp.exp(sc - m)
        den = jnp.sum(e, axis=1, keepdims=True)
        p = e / den
        cols.append(_dot(p, v_ref[:, s]))
    out_ref[...] = jnp.concatenate(cols, axis=1)


def _run_attn(q, k, v, bias, *, interpret=False):
    BQ = 256
    grid = (N // BQ,)
    return pl.pallas_call(
        _attn_body,
        grid=grid,
        in_specs=[
            pl.BlockSpec((BQ, H), lambda i: (i, 0)),
            pl.BlockSpec((N, H), lambda i: (0, 0)),
            pl.BlockSpec((N, H), lambda i: (0, 0)),
            pl.BlockSpec((BQ, N), lambda i: (i, 0)),
        ],
        out_specs=pl.BlockSpec((BQ, H), lambda i: (i, 0)),
        out_shape=jax.ShapeDtypeStruct((N, H), F32),
        interpret=interpret,
    )(q, k, v, bias)


def _combine_body(x_ref, sc_ref, o_ref, w1_ref, b1_ref, w2_ref, b2_ref,
                  etw_ref, wo_ref, bo_ref, f1_ref, f1b_ref, f2_ref, f2b_ref,
                  gamma_ref, beta_ref, out_ref):
    x = x_ref[...]
    aggr = sc_ref[0:ROWS, :] + sc_ref[ROWS:2 * ROWS, :]

    etw = etw_ref[...]
    ew = jnp.exp(etw - jnp.max(etw))
    w = ew / jnp.sum(ew)

    xm = jnp.zeros((N, H), F32)
    for t in range(ET):
        xa = x + aggr[t * N:(t + 1) * N, :]
        h1 = jnp.maximum(_dot_t(xa, w1_ref[t]) + b1_ref[t:t + 1, :], 0.0)
        ht = _dot_t(h1, w2_ref[t]) + b2_ref[t:t + 1, :]
        xm = xm + w[0:1, t:t + 1] * ht

    xat = _dot_t(o_ref[...], wo_ref[...]) + bo_ref[...]
    cat = jnp.concatenate([xm, xat], axis=1)
    h = jnp.maximum(_dot_t(cat, f1_ref[...]) + f1b_ref[...], 0.0)
    f = _dot_t(h, f2_ref[...]) + f2b_ref[...]
    mu = jnp.mean(f, axis=0, keepdims=True)
    d = f - mu
    var = jnp.mean(d * d, axis=0, keepdims=True)
    fn = d * lax.rsqrt(var + 1e-5) * gamma_ref[...] + beta_ref[...]
    out_ref[...] = x + fn


def _run_combine(x, scout, o, w1, b1, w2, b2, etw2, wo, bo2,
                 f1, f1b2, f2, f2b2, gamma2, beta2, *, interpret=False):
    return pl.pallas_call(
        _combine_body,
        out_shape=jax.ShapeDtypeStruct((N, H), F32),
        interpret=interpret,
    )(x, scout, o, w1, b1, w2, b2, etw2, wo, bo2, f1, f1b2, f2, f2b2,
      gamma2, beta2)


def kernel(x, edge_index, edge_attr, edge_type, bias_matrix,
           We, be, W1, b1, W2, b2, etw,
           Wq, bq, Wk, bk, Wv, bv, Wo, bo,
           F1, f1b, F2, f2b, gamma, beta):
    src = edge_index[0]
    dst = edge_index[1]
    tidx = edge_type * N + dst
    srcr = src.reshape(NW, NCHUNK, CHUNK)
    tidxr = tidx.reshape(NW, NCHUNK, CHUNK)

    t24 = jnp.tile(jnp.eye(ED, dtype=F32), (1, ET))
    s24 = jnp.repeat(jnp.eye(ET, dtype=F32), ED, axis=1)
    w24 = We.transpose(0, 2, 1).reshape(ET * ED, H)
    emb = _run_emb(edge_attr, edge_type[:, None], t24, s24, w24, be)
    scout = _run_edge_sc(x, srcr, tidxr, emb)

    wqkv = jnp.concatenate([Wq, Wk, Wv], axis=0)
    bqkv = jnp.concatenate([bq, bk, bv])[None, :]
    qkv = _run_qkv(x, wqkv, bqkv)
    q = qkv[:, 0:H]
    k = qkv[:, H:2 * H]
    v = qkv[:, 2 * H:3 * H]
    o = _run_attn(q, k, v, bias_matrix)

    out = _run_combine(
        x, scout, o, W1, b1, W2, b2, etw[None, :], Wo, bo[None, :],
        F1, f1b[None, :], F2, f2b[None, :], gamma[None, :], beta[None, :])
    return out


# speedup vs baseline: 7.1108x; 1.1936x over previous
import jax
import jax.numpy as jnp
from jax import lax
from jax.experimental import pallas as pl
from jax.experimental.pallas import tpu as pltpu
from jax.experimental.pallas import tpu_sc as plsc

N = 2048
E = 262144
H = 128
NH = 8
DH = H // NH
ET = 4
ED = 6

F32 = jnp.float32
BF16 = jnp.bfloat16

NC = 2
NS = 16
NW = NC * NS
EDGES_PER_W = E // NW
CHUNK = 64
NCHUNK = EDGES_PER_W // CHUNK
NPAIR = NCHUNK // 2
ROWS = ET * N
ROWS_PER_TILE = ROWS // NS

_DN = (((1,), (1,)), ((), ()))
_DN_STD = (((1,), (0,)), ((), ()))


def _dot_t(a, b):
    return lax.dot_general(a, b, _DN, preferred_element_type=F32)


def _dot(a, b):
    return lax.dot_general(a, b, _DN_STD, preferred_element_type=F32)


def _emb_body(attr_ref, et_ref, t24_ref, w28_ref, belo_ref, out_ref):
    attr = attr_ref[...].astype(BF16)
    atiled = _dot(attr, t24_ref[...].astype(BF16))
    et = et_ref[...]
    i24 = lax.broadcasted_iota(jnp.int32, (1, ET * ED), 1) // ED
    m24 = (et == i24).astype(F32)
    feat = (atiled * m24).astype(BF16)
    i4 = lax.broadcasted_iota(jnp.int32, (1, ET), 1)
    oh = (et == i4).astype(F32)
    f28 = jnp.concatenate([feat, oh.astype(BF16)], axis=1)
    out_ref[...] = (_dot(f28, w28_ref[...].astype(BF16))
                    + _dot(oh, belo_ref[...]))


def _run_emb(edge_attr, etcol, t24, w28, belo, *, interpret=False):
    BE = 4096
    grid = (E // BE,)
    return pl.pallas_call(
        _emb_body,
        grid=grid,
        in_specs=[
            pl.BlockSpec((BE, ED), lambda i: (i, 0)),
            pl.BlockSpec((BE, 1), lambda i: (i, 0)),
            pl.BlockSpec((ED, ET * ED), lambda i: (0, 0)),
            pl.BlockSpec((ET * ED + ET, H), lambda i: (0, 0)),
            pl.BlockSpec((ET, H), lambda i: (0, 0)),
        ],
        out_specs=pl.BlockSpec((BE, H), lambda i: (i, 0)),
        out_shape=jax.ShapeDtypeStruct((E, H), F32),
        interpret=interpret,
    )(edge_attr, etcol, t24, w28, belo)


def _relu_add_chunk(emb_v, xr_v):
    UN = 4

    def _grp(i, _):
        for u in range(UN):
            j = i * UN + u
            for k in range(H // 16):
                s = pl.ds(k * 16, 16)
                emb_v[j, s] = jnp.maximum(emb_v[j, s] + xr_v[j, s], 0.0)
        return 0
    lax.fori_loop(0, CHUNK // UN, _grp, 0)


def _edge_sc_body(x_hbm, srcr_hbm, tidxr_hbm, emb_hbm, out_hbm,
                  src_a, tidx_a, tstage, emb0, emb1, xr0, xr1, aggr_s,
                  se0, se1, sg0, sg1, ss0, ss1):
    cid = lax.axis_index("c")
    sid = lax.axis_index("s")
    wid = sid * NC + cid
    base = wid * EDGES_PER_W

    pltpu.sync_copy(srcr_hbm.at[pl.ds(base, EDGES_PER_W)], src_a)
    pltpu.sync_copy(tidxr_hbm.at[pl.ds(base, EDGES_PER_W)], tidx_a)

    def _z(j, _):
        for k in range(H // 16):
            emb0[j, pl.ds(k * 16, 16)] = jnp.zeros((16,), F32)
        return 0
    lax.fori_loop(0, CHUNK, _z, 0)
    for r in range(ROWS_PER_TILE // CHUNK):
        pltpu.sync_copy(emb0, aggr_s.at[pl.ds(sid * ROWS_PER_TILE + r * CHUNK, CHUNK)])
    plsc.subcore_barrier()

    def _start(g, ebuf, xbuf, esem, gsem):
        off = pl.multiple_of(base + g * CHUNK, CHUNK)
        ce = pltpu.async_copy(emb_hbm.at[pl.ds(off, CHUNK)], ebuf, esem)
        cg = pltpu.async_copy(x_hbm.at[src_a.at[pl.ds(g * CHUNK, CHUNK)]],
                              xbuf, gsem)
        del ce, cg

    def _scatter(g, b, ebuf, ssem):
        for k in range(CHUNK // 16):
            tstage[b, pl.ds(k * 16, 16)] = tidx_a[pl.ds(g * CHUNK + k * 16, 16)]
        c = pltpu.async_copy(ebuf, aggr_s.at[tstage.at[b]], ssem, add=True)
        del c

    _start(0, emb0, xr0, se0, sg0)
    _start(1, emb1, xr1, se1, sg1)

    def _pair(p, _):
        g0 = p * 2

        pltpu.make_async_copy(emb_hbm.at[pl.ds(0, CHUNK)], emb0, se0).wait()
        pltpu.make_async_copy(x_hbm.at[src_a.at[pl.ds(0, CHUNK)]], xr0, sg0).wait()
        _relu_add_chunk(emb0, xr0)
        _scatter(g0, 0, emb0, ss0)

        pltpu.make_async_copy(emb_hbm.at[pl.ds(0, CHUNK)], emb1, se1).wait()
        pltpu.make_async_copy(x_hbm.at[src_a.at[pl.ds(0, CHUNK)]], xr1, sg1).wait()
        _relu_add_chunk(emb1, xr1)
        _scatter(g0 + 1, 1, emb1, ss1)

        @pl.when(p < NPAIR - 1)
        def _():
            pltpu.make_async_copy(emb0, aggr_s.at[tstage.at[0]], ss0).wait()
            _start(g0 + 2, emb0, xr0, se0, sg0)
            pltpu.make_async_copy(emb1, aggr_s.at[tstage.at[1]], ss1).wait()
            _start(g0 + 3, emb1, xr1, se1, sg1)
        return 0

    lax.fori_loop(0, NPAIR, _pair, 0)
    pltpu.make_async_copy(emb0, aggr_s.at[tstage.at[0]], ss0).wait()
    pltpu.make_async_copy(emb1, aggr_s.at[tstage.at[1]], ss1).wait()
    plsc.subcore_barrier()

    row0 = cid * ROWS + sid * ROWS_PER_TILE
    pltpu.sync_copy(aggr_s.at[pl.ds(sid * ROWS_PER_TILE, ROWS_PER_TILE)],
                    out_hbm.at[pl.ds(row0, ROWS_PER_TILE)])


def _run_edge_sc(x, srcr, tidxr, emb):
    mesh = plsc.VectorSubcoreMesh(core_axis_name="c", subcore_axis_name="s")
    fn = pl.kernel(
        _edge_sc_body,
        mesh=mesh,
        out_type=jax.ShapeDtypeStruct((NC * ROWS, H), F32),
        scratch_types=[
            pltpu.VMEM((EDGES_PER_W,), jnp.int32),
            pltpu.VMEM((EDGES_PER_W,), jnp.int32),
            pltpu.VMEM((2, CHUNK), jnp.int32),
            pltpu.VMEM((CHUNK, H), F32),
            pltpu.VMEM((CHUNK, H), F32),
            pltpu.VMEM((CHUNK, H), F32),
            pltpu.VMEM((CHUNK, H), F32),
            pltpu.VMEM_SHARED((ROWS, H), F32),
            pltpu.SemaphoreType.DMA,
            pltpu.SemaphoreType.DMA,
            pltpu.SemaphoreType.DMA,
            pltpu.SemaphoreType.DMA,
            pltpu.SemaphoreType.DMA,
            pltpu.SemaphoreType.DMA,
        ],
    )
    return fn(x, srcr, tidxr, emb)


def _qkv_body(x_ref, w_ref, b_ref, out_ref):
    out_ref[...] = _dot_t(x_ref[...], w_ref[...]) + b_ref[...]


def _run_qkv(x, wqkv, bqkv, *, interpret=False):
    return pl.pallas_call(
        _qkv_body,
        out_shape=jax.ShapeDtypeStruct((N, 3 * H), F32),
        interpret=interpret,
    )(x, wqkv, bqkv)


def _attn_body(q_ref, k_ref, v_ref, bias_ref, out_ref):
    bias = bias_ref[...]
    cols = []
    for h in range(NH):
        s = slice(h * DH, (h + 1) * DH)
        qh = q_ref[:, s] * (1.0 / (DH ** 0.5))
        sc = _dot_t(qh, k_ref[:, s]) + bias
        m = jnp.max(sc, axis=1, keepdims=True)
        e = jnp.exp(sc - m)
        den = jnp.sum(e, axis=1, keepdims=True)
        p = e / den
        cols.append(_dot(p, v_ref[:, s]))
    out_ref[...] = jnp.concatenate(cols, axis=1)


def _run_attn(q, k, v, bias, *, interpret=False):
    BQ = 256
    grid = (N // BQ,)
    return pl.pallas_call(
        _attn_body,
        grid=grid,
        in_specs=[
            pl.BlockSpec((BQ, H), lambda i: (i, 0)),
            pl.BlockSpec((N, H), lambda i: (0, 0)),
            pl.BlockSpec((N, H), lambda i: (0, 0)),
            pl.BlockSpec((BQ, N), lambda i: (i, 0)),
        ],
        out_specs=pl.BlockSpec((BQ, H), lambda i: (i, 0)),
        out_shape=jax.ShapeDtypeStruct((N, H), F32),
        interpret=interpret,
    )(q, k, v, bias)


def _combine_body(x_ref, sc_ref, o_ref, w1_ref, b1_ref, w2_ref, b2_ref,
                  etw_ref, wo_ref, bo_ref, f1_ref, f1b_ref, f2_ref, f2b_ref,
                  gamma_ref, beta_ref, out_ref):
    x = x_ref[...]
    aggr = sc_ref[0:ROWS, :] + sc_ref[ROWS:2 * ROWS, :]

    etw = etw_ref[...]
    ew = jnp.exp(etw - jnp.max(etw))
    w = ew / jnp.sum(ew)

    xm = jnp.zeros((N, H), F32)
    for t in range(ET):
        xa = x + aggr[t * N:(t + 1) * N, :]
        h1 = jnp.maximum(_dot_t(xa, w1_ref[t]) + b1_ref[t:t + 1, :], 0.0)
        ht = _dot_t(h1, w2_ref[t]) + b2_ref[t:t + 1, :]
        xm = xm + w[0:1, t:t + 1] * ht

    xat = _dot_t(o_ref[...], wo_ref[...]) + bo_ref[...]
    cat = jnp.concatenate([xm, xat], axis=1)
    h = jnp.maximum(_dot_t(cat, f1_ref[...]) + f1b_ref[...], 0.0)
    f = _dot_t(h, f2_ref[...]) + f2b_ref[...]
    mu = jnp.mean(f, axis=0, keepdims=True)
    d = f - mu
    var = jnp.mean(d * d, axis=0, keepdims=True)
    fn = d * lax.rsqrt(var + 1e-5) * gamma_ref[...] + beta_ref[...]
    out_ref[...] = x + fn


def _run_combine(x, scout, o, w1, b1, w2, b2, etw2, wo, bo2,
                 f1, f1b2, f2, f2b2, gamma2, beta2, *, interpret=False):
    return pl.pallas_call(
        _combine_body,
        out_shape=jax.ShapeDtypeStruct((N, H), F32),
        interpret=interpret,
    )(x, scout, o, w1, b1, w2, b2, etw2, wo, bo2, f1, f1b2, f2, f2b2,
      gamma2, beta2)


def kernel(x, edge_index, edge_attr, edge_type, bias_matrix,
           We, be, W1, b1, W2, b2, etw,
           Wq, bq, Wk, bk, Wv, bv, Wo, bo,
           F1, f1b, F2, f2b, gamma, beta):
    src = edge_index[0]
    dst = edge_index[1]
    tidx = edge_type * N + dst

    t24 = jnp.tile(jnp.eye(ED, dtype=F32), (1, ET))
    w24 = We.transpose(0, 2, 1).reshape(ET * ED, H)
    w28 = jnp.concatenate([w24, be], axis=0)
    belo = be - be.astype(BF16).astype(F32)
    emb = _run_emb(edge_attr, edge_type[:, None], t24, w28, belo)
    scout = _run_edge_sc(x, src, tidx, emb)

    wqkv = jnp.concatenate([Wq, Wk, Wv], axis=0)
    bqkv = jnp.concatenate([bq, bk, bv])[None, :]
    qkv = _run_qkv(x, wqkv, bqkv)
    q = qkv[:, 0:H]
    k = qkv[:, H:2 * H]
    v = qkv[:, 2 * H:3 * H]
    o = _run_attn(q, k, v, bias_matrix)

    out = _run_combine(
        x, scout, o, W1, b1, W2, b2, etw[None, :], Wo, bo[None, :],
        F1, f1b[None, :], F2, f2b[None, :], gamma[None, :], beta[None, :])
    return out
